# baseline (speedup 1.0000x reference)
import functools
import math

import jax
import jax.numpy as jnp
import numpy as np
from jax import lax
from jax.experimental import pallas as pl
from jax.experimental.pallas import tpu as pltpu
from jax.experimental.pallas import tpu_sc as plsc

VOCAB = 100000
SEQ_LEN = 2048
EMB = 128
BATCH = 4

NC = 2
NS = 16
NW = NC * NS
ROWS = BATCH * SEQ_LEN
BPW = ROWS // NW
WPB = NW // BATCH
CH = 128
NCH = BPW // CH
LANES = 16
SCALE = np.float32(math.sqrt(EMB))

_mesh = plsc.VectorSubcoreMesh(core_axis_name="c", subcore_axis_name="s")


@functools.partial(
    pl.kernel,
    out_type=jax.ShapeDtypeStruct((BATCH, SEQ_LEN, EMB), jnp.float32),
    mesh=_mesh,
    scratch_types=[
        pltpu.VMEM((BPW,), jnp.int32),
        pltpu.VMEM((BPW, EMB), jnp.float32),
        pltpu.SemaphoreType.DMA,
        pltpu.SemaphoreType.DMA,
        pltpu.SemaphoreType.DMA,
        pltpu.SemaphoreType.DMA,
        pltpu.SemaphoreType.DMA,
        pltpu.SemaphoreType.DMA,
    ],
)
def _emb_kernel(idx_hbm, tok_hbm, pos_hbm, out_hbm, idx_v, rows_v,
                sem_i, p0, p1, g0, g1, sem_w):
    wid = lax.axis_index("s") * NC + lax.axis_index("c")
    b = wid // WPB
    s0 = lax.rem(wid, WPB) * BPW

    psems = (p0, p1)
    gsems = (g0, g1)

    c_idx = pltpu.async_copy(idx_hbm.at[b].at[pl.ds(s0, BPW)], idx_v, sem_i)
    pcopies = [
        pltpu.async_copy(
            pos_hbm.at[pl.ds(s0 + j * CH, CH)],
            rows_v.at[pl.ds(j * CH, CH)],
            psems[j],
        )
        for j in range(NCH)
    ]
    c_idx.wait()

    gcopies = []
    for j in range(NCH):
        pcopies[j].wait()
        gcopies.append(
            pltpu.async_copy(
                tok_hbm.at[idx_v.at[pl.ds(j * CH, CH)]],
                rows_v.at[pl.ds(j * CH, CH)],
                gsems[j],
                add=True,
            )
        )

    QR = CH // 2

    def compute_part(base_row):
        @plsc.parallel_loop(0, QR, 1, unroll=2)
        def body(i):
            row = base_row + i
            for k in range(EMB // LANES):
                sl = pl.ds(k * LANES, LANES)
                rows_v[row, sl] = rows_v[row, sl] * SCALE

    writes = []
    for j in range(NCH):
        gcopies[j].wait()
        for q in range(CH // QR):
            r0 = j * CH + q * QR
            compute_part(r0)
            writes.append(
                pltpu.async_copy(
                    rows_v.at[pl.ds(r0, QR)],
                    out_hbm.at[b].at[pl.ds(s0 + r0, QR)],
                    sem_w,
                )
            )
    for w in writes:
        w.wait()


def kernel(x_bs, tok_weight, pos_weight):
    return _emb_kernel(x_bs, tok_weight, pos_weight)
